# baseline (speedup 1.0000x reference)
import jax
import jax.numpy as jnp
from jax.experimental import pallas as pl
from jax.experimental.pallas import tpu as pltpu

MAX_NUM_TILES = 4
HIDDEN_SIZE = 1280
NUM_PATCHES = 1025


def _add_kernel(ids_ref, h_ref, emb_ref, gate_ref, out_ref):
    g = jnp.tanh(gate_ref[0, 0])
    out_ref[...] = h_ref[...] + g * emb_ref[0][None, None, :, :]


def kernel(hidden_state, aspect_ratio_ids, embedding_table, gate):
    batch = hidden_state.shape[0]
    ids = aspect_ratio_ids.astype(jnp.int32)
    gate2d = gate.reshape(1, 1)
    table3d = embedding_table.reshape(-1, 1, HIDDEN_SIZE)

    grid_spec = pltpu.PrefetchScalarGridSpec(
        num_scalar_prefetch=1,
        grid=(batch, MAX_NUM_TILES),
        in_specs=[
            pl.BlockSpec(
                (1, 1, NUM_PATCHES, HIDDEN_SIZE),
                lambda b, t, ids: (b, t, 0, 0),
            ),
            pl.BlockSpec(
                (1, 1, HIDDEN_SIZE),
                lambda b, t, ids: (ids[b] * MAX_NUM_TILES + t, 0, 0),
            ),
            pl.BlockSpec((1, 1), lambda b, t, ids: (0, 0)),
        ],
        out_specs=pl.BlockSpec(
            (1, 1, NUM_PATCHES, HIDDEN_SIZE),
            lambda b, t, ids: (b, t, 0, 0),
        ),
    )

    return pl.pallas_call(
        _add_kernel,
        grid_spec=grid_spec,
        out_shape=jax.ShapeDtypeStruct(hidden_state.shape, hidden_state.dtype),
    )(ids, hidden_state, table3d, gate2d)


# speedup vs baseline: 1.0000x; 1.0000x over previous
import jax
import jax.numpy as jnp
from jax.experimental import pallas as pl
from jax.experimental.pallas import tpu as pltpu

MAX_NUM_TILES = 4
HIDDEN_SIZE = 1280
NUM_PATCHES = 1025


def _add_kernel(ids_ref, h_ref, emb_ref, gate_ref, out_ref):
    g = jnp.tanh(gate_ref[0, 0])
    out_ref[...] = h_ref[...] + g * emb_ref[0][None, None, :, :]


def kernel(hidden_state, aspect_ratio_ids, embedding_table, gate):
    batch = hidden_state.shape[0]
    ids = aspect_ratio_ids.astype(jnp.int32)
    gate2d = gate.reshape(1, 1)
    table3d = embedding_table.reshape(-1, 1, HIDDEN_SIZE)

    grid_spec = pltpu.PrefetchScalarGridSpec(
        num_scalar_prefetch=1,
        grid=(batch, MAX_NUM_TILES),
        in_specs=[
            pl.BlockSpec(
                (1, 1, NUM_PATCHES, HIDDEN_SIZE),
                lambda b, t, ids: (b, t, 0, 0),
            ),
            pl.BlockSpec(
                (1, 1, HIDDEN_SIZE),
                lambda b, t, ids: (ids[b] * MAX_NUM_TILES + t, 0, 0),
            ),
            pl.BlockSpec((1, 1), lambda b, t, ids: (0, 0)),
        ],
        out_specs=pl.BlockSpec(
            (1, 1, NUM_PATCHES, HIDDEN_SIZE),
            lambda b, t, ids: (b, t, 0, 0),
        ),
    )

    return pl.pallas_call(
        _add_kernel,
        grid_spec=grid_spec,
        out_shape=jax.ShapeDtypeStruct(hidden_state.shape, hidden_state.dtype),
        compiler_params=pltpu.CompilerParams(
            dimension_semantics=("parallel", "parallel"),
        ),
    )(ids, hidden_state, table3d, gate2d)
